# Initial kernel scaffold; baseline (speedup 1.0000x reference)
import jax
import jax.numpy as jnp
from jax.experimental import pallas as pl


def kernel(x, rel_table):
    raise NotImplementedError("write your pallas kernel here")


# speedup vs baseline: 192.1465x; 192.1465x over previous
import jax
import jax.numpy as jnp
from jax.experimental import pallas as pl
from jax.experimental.pallas import tpu as pltpu


def _body(slab_ref, x_ref, o_ref, avg_ref):
    S, D = avg_ref.shape

    @pl.when(pl.program_id(0) == 0)
    def _():
        i = jax.lax.broadcasted_iota(jnp.int32, (S, 2 * S), 0)
        k = jax.lax.broadcasted_iota(jnp.int32, (S, 2 * S), 1)
        w = ((k >= (S - 1) - i) & (k <= 2 * (S - 1) - i)).astype(jnp.float32)
        avg_ref[...] = jnp.dot(
            w, slab_ref[...], preferred_element_type=jnp.float32
        ) * (1.0 / S)

    o_ref[...] = x_ref[...] + avg_ref[...][None]


def kernel(x, rel_table):
    B, S, D = x.shape
    max_len = (rel_table.shape[0] + 1) // 2
    lo = max_len - S
    slab = jax.lax.slice(rel_table, (lo, 0), (lo + 2 * S - 1, D))
    slab = jnp.pad(slab, ((0, 1), (0, 0)))
    return pl.pallas_call(
        _body,
        grid=(B,),
        in_specs=[
            pl.BlockSpec((2 * S, D), lambda b: (0, 0)),
            pl.BlockSpec((1, S, D), lambda b: (b, 0, 0)),
        ],
        out_specs=pl.BlockSpec((1, S, D), lambda b: (b, 0, 0)),
        out_shape=jax.ShapeDtypeStruct((B, S, D), jnp.float32),
        scratch_shapes=[pltpu.VMEM((S, D), jnp.float32)],
    )(slab, x)
